# Initial kernel scaffold; baseline (speedup 1.0000x reference)
import functools

import jax
import jax.numpy as jnp
from jax import lax
from jax.experimental import pallas as pl
from jax.experimental.pallas import tpu as pltpu
from jax.experimental.pallas import tpu_sc as plsc

VOCAB = 1000
NC = 2
NS = 16
NW = NC * NS


def _make_gather(bt, k):
    b_per_w = bt // NW
    nchunk = b_per_w // k
    mesh = plsc.VectorSubcoreMesh(core_axis_name="c", subcore_axis_name="s")

    @functools.partial(
        pl.kernel,
        out_type=jax.ShapeDtypeStruct((bt, VOCAB), jnp.float32),
        mesh=mesh,
        scratch_types=[
            pltpu.VMEM((b_per_w,), jnp.int32),
            pltpu.VMEM((k, VOCAB), jnp.float32),
            pltpu.SemaphoreType.DMA,
        ],
    )
    def gather_kernel(table_hbm, idx_hbm, out_hbm, idx_v, rows_v, sem):
        wid = lax.axis_index("s") * NC + lax.axis_index("c")
        base = wid * b_per_w
        pltpu.sync_copy(idx_hbm.at[pl.ds(base, b_per_w)], idx_v)

        def body(c, carry):
            off = c * k
            pltpu.async_copy(
                table_hbm.at[idx_v.at[pl.ds(off, k)]], rows_v, sem
            ).wait()
            pltpu.sync_copy(rows_v, out_hbm.at[pl.ds(base + off, k)])
            return carry

        lax.fori_loop(0, nchunk, body, 0)

    return gather_kernel


_gather = _make_gather(81920, 64)


@jax.jit
def kernel(idx, token_embedding_table):
    b, t = idx.shape
    flat = idx.reshape(b * t)
    out = _gather(token_embedding_table, flat)
    return out.reshape(b, t, VOCAB)


# speedup vs baseline: 1.4077x; 1.4077x over previous
import functools

import jax
import jax.numpy as jnp
from jax import lax
from jax.experimental import pallas as pl
from jax.experimental.pallas import tpu as pltpu
from jax.experimental.pallas import tpu_sc as plsc

VOCAB = 1000
NC = 2
NS = 16
NW = NC * NS


def _make_gather(bt, k):
    b_per_w = bt // NW
    nchunk = b_per_w // k
    mesh = plsc.VectorSubcoreMesh(core_axis_name="c", subcore_axis_name="s")

    @functools.partial(
        pl.kernel,
        out_type=jax.ShapeDtypeStruct((bt, VOCAB), jnp.float32),
        mesh=mesh,
        scratch_types=[
            pltpu.VMEM((b_per_w,), jnp.int32),
            pltpu.VMEM((k, VOCAB), jnp.float32),
            pltpu.SemaphoreType.DMA,
        ],
        compiler_params=pltpu.CompilerParams(use_tc_tiling_on_sc=False),
    )
    def gather_kernel(table_hbm, idx_hbm, out_hbm, idx_v, rows_v, sem):
        wid = lax.axis_index("s") * NC + lax.axis_index("c")
        base = wid * b_per_w
        pltpu.sync_copy(idx_hbm.at[pl.ds(base, b_per_w)], idx_v)

        def body(c, carry):
            off = c * k
            pltpu.async_copy(
                table_hbm.at[idx_v.at[pl.ds(off, k)]], rows_v, sem
            ).wait()
            pltpu.sync_copy(rows_v, out_hbm.at[pl.ds(base + off, k)])
            return carry

        lax.fori_loop(0, nchunk, body, 0)

    return gather_kernel


_gather = _make_gather(81920, 64)


@jax.jit
def kernel(idx, token_embedding_table):
    b, t = idx.shape
    flat = idx.reshape(b * t)
    out = _gather(token_embedding_table, flat)
    return out.reshape(b, t, VOCAB)


# speedup vs baseline: 1.4408x; 1.0235x over previous
import functools

import jax
import jax.numpy as jnp
from jax import lax
from jax.experimental import pallas as pl
from jax.experimental.pallas import tpu as pltpu
from jax.experimental.pallas import tpu_sc as plsc

VOCAB = 1000
NC = 2
NS = 16
NW = NC * NS


def _make_gather(bt, k):
    b_per_w = bt // NW
    nchunk = b_per_w // k
    assert nchunk % 2 == 0 and nchunk >= 4
    mesh = plsc.VectorSubcoreMesh(core_axis_name="c", subcore_axis_name="s")

    @functools.partial(
        pl.kernel,
        out_type=jax.ShapeDtypeStruct((bt, VOCAB), jnp.float32),
        mesh=mesh,
        scratch_types=[
            pltpu.VMEM((b_per_w,), jnp.int32),
            pltpu.VMEM((2, k, VOCAB), jnp.float32),
            pltpu.SemaphoreType.DMA,
            pltpu.SemaphoreType.DMA,
        ],
        compiler_params=pltpu.CompilerParams(use_tc_tiling_on_sc=False),
    )
    def gather_kernel(table_hbm, idx_hbm, out_hbm, idx_v, rows_v, sem0, sem1):
        wid = lax.axis_index("s") * NC + lax.axis_index("c")
        base = wid * b_per_w
        sems = (sem0, sem1)
        pltpu.sync_copy(idx_hbm.at[pl.ds(base, b_per_w)], idx_v)

        def gather_dma(c, slot):
            return pltpu.make_async_copy(
                table_hbm.at[idx_v.at[pl.ds(c * k, k)]],
                rows_v.at[slot],
                sems[slot],
            )

        def out_copy(c, slot):
            pltpu.sync_copy(rows_v.at[slot], out_hbm.at[pl.ds(base + c * k, k)])

        gather_dma(0, 0).start()

        def body(c2, carry):
            c = 2 * c2
            gather_dma(c + 1, 1).start()
            gather_dma(c, 0).wait()
            out_copy(c, 0)
            gather_dma(c + 2, 0).start()
            gather_dma(c + 1, 1).wait()
            out_copy(c + 1, 1)
            return carry

        lax.fori_loop(0, nchunk // 2 - 1, body, 0)
        c = nchunk - 2
        gather_dma(c + 1, 1).start()
        gather_dma(c, 0).wait()
        out_copy(c, 0)
        gather_dma(c + 1, 1).wait()
        out_copy(c + 1, 1)

    return gather_kernel


_gather = _make_gather(81920, 64)


@jax.jit
def kernel(idx, token_embedding_table):
    b, t = idx.shape
    flat = idx.reshape(b * t)
    out = _gather(token_embedding_table, flat)
    return out.reshape(b, t, VOCAB)
